# Initial kernel scaffold; baseline (speedup 1.0000x reference)
import jax
import jax.numpy as jnp
from jax.experimental import pallas as pl


def kernel(node_features, batch_ids, neigh_ids, W_self, W_comb, bias_comb):
    raise NotImplementedError("write your pallas kernel here")


# speedup vs baseline: 3.1425x; 3.1425x over previous
import functools

import jax
import jax.numpy as jnp
from jax import lax
from jax.experimental import pallas as pl
from jax.experimental.pallas import tpu as pltpu
from jax.experimental.pallas import tpu_sc as plsc

N = 50000
D = 256
B = 8192
DEG = 32
OUT = 256

_NC = 2
_NS = 16
_L = 16
_NW = _NC * _NS
_BW = B // _NW
_C = 2
_G = _C * DEG
_NCH = _BW // _C
_NF_CH = 64


_BM1 = 2000

def _mm_body(x_ref, w_ref, o_ref):
    o_ref[...] = lax.dot_general(
        x_ref[...], w_ref[...], (((1,), (1,)), ((), ())),
        preferred_element_type=jnp.float32)


def _transform(nf, w_self):
    return pl.pallas_call(
        _mm_body,
        grid=(N // _BM1,),
        in_specs=[
            pl.BlockSpec((_BM1, D), lambda i: (i, 0)),
            pl.BlockSpec((D, D), lambda i: (0, 0)),
        ],
        out_specs=pl.BlockSpec((_BM1, D), lambda i: (i, 0)),
        out_shape=jax.ShapeDtypeStruct((N, D), jnp.float32),
    )(nf, w_self)


@functools.partial(
    pl.kernel,
    out_type=[
        jax.ShapeDtypeStruct((B, D), jnp.float32),
        jax.ShapeDtypeStruct((B, D), jnp.float32),
    ],
    mesh=plsc.VectorSubcoreMesh(core_axis_name="c", subcore_axis_name="s"),
    scratch_types=[
        pltpu.VMEM((_BW * DEG,), jnp.int32),
        pltpu.VMEM((_BW,), jnp.int32),
        pltpu.VMEM((_NF_CH, D), jnp.float32),
        pltpu.VMEM((_G, D), jnp.float32),
        pltpu.VMEM((_G, D), jnp.float32),
        pltpu.VMEM((_BW, D), jnp.float32),
        pltpu.SemaphoreType.DMA,
        pltpu.SemaphoreType.DMA,
        pltpu.SemaphoreType.DMA,
    ],
)
def _sc_gather_max(trans_hbm, nf_hbm, bids_hbm, nids_hbm, agg_out, nfs_out,
                   nidx_v, bidx_v, nf_buf, g0, g1, agg_v, sem_n, sem0, sem1):
    wid = lax.axis_index("s") * _NC + lax.axis_index("c")
    base = wid * _BW

    pltpu.sync_copy(nids_hbm.at[pl.ds(base * DEG, _BW * DEG)], nidx_v)
    pltpu.sync_copy(bids_hbm.at[pl.ds(base, _BW)], bidx_v)

    for h in range(_BW // _NF_CH):
        pltpu.async_copy(
            nf_hbm.at[bidx_v.at[pl.ds(h * _NF_CH, _NF_CH)]], nf_buf, sem_n
        ).wait()
        pltpu.sync_copy(nf_buf, nfs_out.at[pl.ds(base + h * _NF_CH, _NF_CH)])

    def gstart(i, buf, sem):
        pltpu.async_copy(trans_hbm.at[nidx_v.at[pl.ds(i * _G, _G)]], buf, sem)

    def gwait(i, buf, sem):
        pltpu.make_async_copy(
            trans_hbm.at[nidx_v.at[pl.ds(i * _G, _G)]], buf, sem).wait()

    def reduce_chunk(i, buf):
        for c in range(_C):
            row = i * _C + c
            for g in range(D // _L):
                sl = pl.ds(g * _L, _L)
                accs = [buf[c * DEG + j, sl] for j in range(4)]
                for j in range(4, DEG):
                    accs[j % 4] = jnp.maximum(accs[j % 4], buf[c * DEG + j, sl])
                agg_v[row, sl] = jnp.maximum(
                    jnp.maximum(accs[0], accs[1]),
                    jnp.maximum(accs[2], accs[3]))

    gstart(0, g0, sem0)

    def body(k, carry):
        a = 2 * k
        b = a + 1
        gstart(b, g1, sem1)
        gwait(a, g0, sem0)
        reduce_chunk(a, g0)

        @pl.when(b + 1 < _NCH)
        def _():
            gstart(b + 1, g0, sem0)

        gwait(b, g1, sem1)
        reduce_chunk(b, g1)
        return carry

    lax.fori_loop(0, _NCH // 2, body, 0)

    pltpu.sync_copy(agg_v, agg_out.at[pl.ds(base, _BW)])


_BM3 = 1024

def _fin_body(nf_ref, ag_ref, w1_ref, w2_ref, b_ref, o_ref):
    acc = lax.dot_general(
        nf_ref[...], w1_ref[...], (((1,), (1,)), ((), ())),
        preferred_element_type=jnp.float32)
    acc = acc + lax.dot_general(
        ag_ref[...], w2_ref[...], (((1,), (1,)), ((), ())),
        preferred_element_type=jnp.float32)
    o_ref[...] = acc + b_ref[...]


def _final(nfeats, agg, w1, w2, bias):
    return pl.pallas_call(
        _fin_body,
        grid=(B // _BM3,),
        in_specs=[
            pl.BlockSpec((_BM3, D), lambda i: (i, 0)),
            pl.BlockSpec((_BM3, D), lambda i: (i, 0)),
            pl.BlockSpec((OUT, D), lambda i: (0, 0)),
            pl.BlockSpec((OUT, D), lambda i: (0, 0)),
            pl.BlockSpec((1, OUT), lambda i: (0, 0)),
        ],
        out_specs=pl.BlockSpec((_BM3, OUT), lambda i: (i, 0)),
        out_shape=jax.ShapeDtypeStruct((B, OUT), jnp.float32),
    )(nfeats, agg, w1, w2, bias)


@jax.jit
def kernel(node_features, batch_ids, neigh_ids, W_self, W_comb, bias_comb):
    transformed = _transform(node_features, W_self)
    agg, nfeats = _sc_gather_max(
        transformed, node_features, batch_ids, neigh_ids.reshape(-1))
    w1 = W_comb[:, :D]
    w2 = W_comb[:, D:]
    return _final(nfeats, agg, w1, w2, bias_comb.reshape(1, OUT))
